# Initial kernel scaffold; baseline (speedup 1.0000x reference)
import jax
import jax.numpy as jnp
from jax.experimental import pallas as pl


def kernel(skills, hardness, position_embed, skill_embed, hardness_embed):
    raise NotImplementedError("write your pallas kernel here")


# speedup vs baseline: 1.8720x; 1.8720x over previous
import functools

import jax
import jax.numpy as jnp
from jax import lax
from jax.experimental import pallas as pl
from jax.experimental.pallas import tpu as pltpu
from jax.experimental.pallas import tpu_sc as plsc

D = 64
T = 200
B = 4096
N = B * T
NW = 32
RPW = N // NW
CH = 128
NCH = RPW // CH
LANES = 16

_mesh = plsc.VectorSubcoreMesh(core_axis_name="c", subcore_axis_name="s")


@functools.partial(
    pl.kernel,
    mesh=_mesh,
    out_type=jax.ShapeDtypeStruct((N, D), jnp.float32),
    compiler_params=pltpu.CompilerParams(use_tc_tiling_on_sc=False),
    scratch_types=[
        pltpu.VMEM((NCH, CH), jnp.int32),
        pltpu.VMEM((NCH, CH), jnp.int32),
        pltpu.VMEM((T, D), jnp.float32),
        pltpu.VMEM((CH, D), jnp.float32),
        pltpu.VMEM((CH, D), jnp.float32),
        pltpu.SemaphoreType.DMA,
        pltpu.SemaphoreType.DMA,
    ],
)
def _embed_sc(skills_hbm, hardness_hbm, pos_hbm, skill_emb_hbm, hard_emb_hbm,
              out_hbm, idx_s, idx_h, pos_v, buf_s, buf_h, sem_s, sem_h):
    wid = lax.axis_index("s") * 2 + lax.axis_index("c")
    row0 = wid * RPW

    pltpu.sync_copy(skills_hbm.at[pl.ds(wid * NCH, NCH)], idx_s)
    pltpu.sync_copy(hardness_hbm.at[pl.ds(wid * NCH, NCH)], idx_h)
    pltpu.sync_copy(pos_hbm, pos_v)

    def chunk_body(c, carry):
        cp_s = pltpu.async_copy(skill_emb_hbm.at[idx_s.at[c]], buf_s, sem_s)
        cp_h = pltpu.async_copy(hard_emb_hbm.at[idx_h.at[c]], buf_h, sem_h)
        cp_s.wait()
        cp_h.wait()
        t0 = lax.rem(c * CH, T)

        def row_body(i, carry2):
            t = lax.rem(t0 + i, T)
            for j in range(D // LANES):
                sl = pl.ds(j * LANES, LANES)
                buf_s[i, sl] = buf_s[i, sl] + buf_h[i, sl] + pos_v[t, sl]
            return carry2

        lax.fori_loop(0, CH, row_body, 0)
        pltpu.sync_copy(buf_s, out_hbm.at[pl.ds(row0 + c * CH, CH)])
        return carry

    lax.fori_loop(0, NCH, chunk_body, 0)


def kernel(skills, hardness, position_embed, skill_embed, hardness_embed):
    skills_r = skills.reshape(N // CH, CH).astype(jnp.int32)
    hardness_r = hardness.reshape(N // CH, CH).astype(jnp.int32)
    out = _embed_sc(skills_r, hardness_r, position_embed, skill_embed,
                    hardness_embed)
    return out.reshape(B, T, D)


# speedup vs baseline: 2.2558x; 1.2050x over previous
import functools

import jax
import jax.numpy as jnp
from jax import lax
from jax.experimental import pallas as pl
from jax.experimental.pallas import tpu as pltpu
from jax.experimental.pallas import tpu_sc as plsc

D = 64
T = 200
B = 4096
N = B * T
NW = 32
RPW = N // NW
CH = 100
NCH = RPW // CH

_mesh = plsc.VectorSubcoreMesh(core_axis_name="c", subcore_axis_name="s")


@functools.partial(
    pl.kernel,
    mesh=_mesh,
    out_type=jax.ShapeDtypeStruct((N, D), jnp.float32),
    compiler_params=pltpu.CompilerParams(use_tc_tiling_on_sc=False),
    scratch_types=[
        pltpu.VMEM((NCH, CH), jnp.int32),
        pltpu.VMEM((NCH, CH), jnp.int32),
        pltpu.VMEM_SHARED((T, D), jnp.float32),
        pltpu.VMEM((CH, D), jnp.float32),
        pltpu.SemaphoreType.DMA,
        pltpu.SemaphoreType.DMA,
    ],
)
def _embed_sc(skills_hbm, hardness_hbm, pos_hbm, skill_emb_hbm, hard_emb_hbm,
              out_hbm, idx_s, idx_h, pos_sh, buf_o, sem_s, sem_h):
    sid = lax.axis_index("s")
    wid = sid * 2 + lax.axis_index("c")
    row0 = wid * RPW

    pltpu.sync_copy(skills_hbm.at[pl.ds(wid * NCH, NCH)], idx_s)
    pltpu.sync_copy(hardness_hbm.at[pl.ds(wid * NCH, NCH)], idx_h)

    @pl.when(sid == 0)
    def _():
        pltpu.sync_copy(pos_hbm, pos_sh)

    plsc.subcore_barrier()

    def chunk_body(c, carry):
        t0 = lax.rem(c * CH, T)
        pltpu.sync_copy(pos_sh.at[pl.ds(t0, CH)], buf_o)
        cp_s = pltpu.async_copy(skill_emb_hbm.at[idx_s.at[c]], buf_o, sem_s,
                                add=True)
        cp_h = pltpu.async_copy(hard_emb_hbm.at[idx_h.at[c]], buf_o, sem_h,
                                add=True)
        cp_s.wait()
        cp_h.wait()
        pltpu.sync_copy(buf_o, out_hbm.at[pl.ds(row0 + c * CH, CH)])
        return carry

    lax.fori_loop(0, NCH, chunk_body, 0)


def kernel(skills, hardness, position_embed, skill_embed, hardness_embed):
    skills_r = skills.reshape(N // CH, CH).astype(jnp.int32)
    hardness_r = hardness.reshape(N // CH, CH).astype(jnp.int32)
    out = _embed_sc(skills_r, hardness_r, position_embed, skill_embed,
                    hardness_embed)
    return out.reshape(B, T, D)


# speedup vs baseline: 2.5560x; 1.1331x over previous
import functools

import jax
import jax.numpy as jnp
from jax import lax
from jax.experimental import pallas as pl
from jax.experimental.pallas import tpu as pltpu
from jax.experimental.pallas import tpu_sc as plsc

D = 64
T = 200
B = 4096
N = B * T
NW = 32
RPW = N // NW
CH = 100
NCH = RPW // CH

_mesh = plsc.VectorSubcoreMesh(core_axis_name="c", subcore_axis_name="s")


@functools.partial(
    pl.kernel,
    mesh=_mesh,
    out_type=jax.ShapeDtypeStruct((N, D), jnp.float32),
    compiler_params=pltpu.CompilerParams(use_tc_tiling_on_sc=False),
    scratch_types=[
        pltpu.VMEM((NCH, CH), jnp.int32),
        pltpu.VMEM((NCH, CH), jnp.int32),
        pltpu.VMEM_SHARED((T, D), jnp.float32),
        pltpu.VMEM((CH, D), jnp.float32),
        pltpu.VMEM((CH, D), jnp.float32),
        pltpu.SemaphoreType.DMA,
        pltpu.SemaphoreType.DMA,
        pltpu.SemaphoreType.DMA,
        pltpu.SemaphoreType.DMA,
        pltpu.SemaphoreType.DMA,
        pltpu.SemaphoreType.DMA,
    ],
)
def _embed_sc(skills_hbm, hardness_hbm, pos_hbm, skill_emb_hbm, hard_emb_hbm,
              out_hbm, idx_s, idx_h, pos_sh, buf0, buf1,
              sem_s0, sem_h0, sem_o0, sem_s1, sem_h1, sem_o1):
    sid = lax.axis_index("s")
    wid = sid * 2 + lax.axis_index("c")
    row0 = wid * RPW

    bufs = (buf0, buf1)
    sems = ((sem_s0, sem_h0, sem_o0), (sem_s1, sem_h1, sem_o1))

    pltpu.sync_copy(skills_hbm.at[pl.ds(wid * NCH, NCH)], idx_s)
    pltpu.sync_copy(hardness_hbm.at[pl.ds(wid * NCH, NCH)], idx_h)

    @pl.when(sid == 0)
    def _():
        pltpu.sync_copy(pos_hbm, pos_sh)

    plsc.subcore_barrier()

    def init_and_gather(c, buf, sem_s, sem_h):
        t0 = lax.rem(c * CH, T)
        pltpu.sync_copy(pos_sh.at[pl.ds(t0, CH)], buf)
        pltpu.async_copy(skill_emb_hbm.at[idx_s.at[c]], buf, sem_s, add=True)
        pltpu.async_copy(hard_emb_hbm.at[idx_h.at[c]], buf, sem_h, add=True)

    init_and_gather(0, buf0, sem_s0, sem_h0)

    def body(g, carry):
        for b in range(2):
            c = g * 2 + b
            buf_p, (sem_sp, sem_hp, sem_op) = bufs[b], sems[b]
            buf_q, (sem_sq, sem_hq, sem_oq) = bufs[1 - b], sems[1 - b]

            @pl.when(c >= 1)
            def _():
                pltpu.make_async_copy(
                    buf_q, out_hbm.at[pl.ds(row0, CH)], sem_oq).wait()

            @pl.when(c < NCH - 1)
            def _():
                init_and_gather(c + 1, buf_q, sem_sq, sem_hq)

            pltpu.make_async_copy(
                skill_emb_hbm.at[idx_s.at[c]], buf_p, sem_sp).wait()
            pltpu.make_async_copy(
                hard_emb_hbm.at[idx_h.at[c]], buf_p, sem_hp).wait()
            pltpu.async_copy(
                buf_p, out_hbm.at[pl.ds(row0 + c * CH, CH)], sem_op)
        return carry

    lax.fori_loop(0, NCH // 2, body, 0)

    pltpu.make_async_copy(buf1, out_hbm.at[pl.ds(row0, CH)], sem_o1).wait()


def kernel(skills, hardness, position_embed, skill_embed, hardness_embed):
    skills_r = skills.reshape(N // CH, CH).astype(jnp.int32)
    hardness_r = hardness.reshape(N // CH, CH).astype(jnp.int32)
    out = _embed_sc(skills_r, hardness_r, position_embed, skill_embed,
                    hardness_embed)
    return out.reshape(B, T, D)


# speedup vs baseline: 2.6455x; 1.0350x over previous
import functools

import jax
import jax.numpy as jnp
from jax import lax
from jax.experimental import pallas as pl
from jax.experimental.pallas import tpu as pltpu
from jax.experimental.pallas import tpu_sc as plsc

D = 64
T = 200
B = 4096
N = B * T
NW = 32
RPW = N // NW
CH = 128
NCH = RPW // CH
NBUF = 4
LA = 3

_mesh = plsc.VectorSubcoreMesh(core_axis_name="c", subcore_axis_name="s")


@functools.partial(
    pl.kernel,
    mesh=_mesh,
    out_type=jax.ShapeDtypeStruct((N, D), jnp.float32),
    compiler_params=pltpu.CompilerParams(use_tc_tiling_on_sc=False),
    scratch_types=[
        pltpu.VMEM((NCH, CH), jnp.int32),
        pltpu.VMEM((NCH, CH), jnp.int32),
        pltpu.VMEM_SHARED((2 * T, D), jnp.float32),
        [pltpu.VMEM((CH, D), jnp.float32)] * NBUF,
        [pltpu.SemaphoreType.DMA] * NBUF,
        [pltpu.SemaphoreType.DMA] * NBUF,
        [pltpu.SemaphoreType.DMA] * NBUF,
    ],
)
def _embed_sc(skills_hbm, hardness_hbm, pos_hbm, skill_emb_hbm, hard_emb_hbm,
              out_hbm, idx_s, idx_h, pos_sh, bufs, sem_s, sem_h, sem_o):
    sid = lax.axis_index("s")
    wid = sid * 2 + lax.axis_index("c")
    row0 = wid * RPW

    pltpu.sync_copy(skills_hbm.at[pl.ds(wid * NCH, NCH)], idx_s)
    pltpu.sync_copy(hardness_hbm.at[pl.ds(wid * NCH, NCH)], idx_h)

    @pl.when(sid == 0)
    def _():
        pltpu.sync_copy(pos_hbm, pos_sh.at[pl.ds(0, T)])
        pltpu.sync_copy(pos_hbm, pos_sh.at[pl.ds(T, T)])

    plsc.subcore_barrier()

    def init_and_gather(c, b):
        t0 = lax.rem(c * CH, T)
        pltpu.sync_copy(pos_sh.at[pl.ds(t0, CH)], bufs[b])
        pltpu.async_copy(skill_emb_hbm.at[idx_s.at[c]], bufs[b], sem_s[b],
                         add=True)
        pltpu.async_copy(hard_emb_hbm.at[idx_h.at[c]], bufs[b], sem_h[b],
                         add=True)

    for k in range(LA):
        init_and_gather(k, k)

    def body(g, carry):
        for b in range(NBUF):
            c = g * NBUF + b
            bn = (b + LA) % NBUF

            @pl.when(c + LA - NBUF >= 0)
            def _():
                pltpu.make_async_copy(
                    bufs[bn], out_hbm.at[pl.ds(row0, CH)], sem_o[bn]).wait()

            @pl.when(c + LA < NCH)
            def _():
                init_and_gather(c + LA, bn)

            pltpu.make_async_copy(
                skill_emb_hbm.at[idx_s.at[c]], bufs[b], sem_s[b]).wait()
            pltpu.make_async_copy(
                hard_emb_hbm.at[idx_h.at[c]], bufs[b], sem_h[b]).wait()
            pltpu.async_copy(
                bufs[b], out_hbm.at[pl.ds(row0 + c * CH, CH)], sem_o[b])
        return carry

    lax.fori_loop(0, NCH // NBUF, body, 0)

    bl = (NCH - 1) % NBUF
    pltpu.make_async_copy(bufs[bl], out_hbm.at[pl.ds(row0, CH)],
                          sem_o[bl]).wait()


def kernel(skills, hardness, position_embed, skill_embed, hardness_embed):
    skills_r = skills.reshape(N // CH, CH).astype(jnp.int32)
    hardness_r = hardness.reshape(N // CH, CH).astype(jnp.int32)
    out = _embed_sc(skills_r, hardness_r, position_embed, skill_embed,
                    hardness_embed)
    return out.reshape(B, T, D)


# speedup vs baseline: 3.4242x; 1.2944x over previous
import functools

import jax
import jax.numpy as jnp
from jax import lax
from jax.experimental import pallas as pl
from jax.experimental.pallas import tpu as pltpu
from jax.experimental.pallas import tpu_sc as plsc

D = 64
T = 200
B = 4096
N = B * T
NW = 32
RPW = N // NW
CH = 128
NCH = RPW // CH
NBUF = 4
LA = 3

_mesh = plsc.VectorSubcoreMesh(core_axis_name="c", subcore_axis_name="s")


@functools.partial(
    pl.kernel,
    mesh=_mesh,
    out_type=jax.ShapeDtypeStruct((N, 2 * D), jnp.float32),
    compiler_params=pltpu.CompilerParams(use_tc_tiling_on_sc=False),
    scratch_types=[
        pltpu.VMEM((NCH, CH), jnp.int32),
        pltpu.VMEM((NCH, CH), jnp.int32),
        pltpu.VMEM_SHARED((2 * T, D), jnp.float32),
        [pltpu.VMEM((CH, D), jnp.float32)] * NBUF,
        [pltpu.SemaphoreType.DMA] * NBUF,
        [pltpu.SemaphoreType.DMA] * NBUF,
        [pltpu.SemaphoreType.DMA] * NBUF,
    ],
)
def _embed_sc(skills_hbm, hardness_hbm, pos_hbm, skill_emb_hbm, hard_emb_hbm,
              out_hbm, idx_s, idx_h, pos_sh, bufs, sem_s, sem_h, sem_o):
    sid = lax.axis_index("s")
    wid = sid * 2 + lax.axis_index("c")
    row0 = wid * RPW

    pltpu.sync_copy(skills_hbm.at[pl.ds(wid * NCH, NCH)], idx_s)
    pltpu.sync_copy(hardness_hbm.at[pl.ds(wid * NCH, NCH)], idx_h)

    @pl.when(sid == 0)
    def _():
        pltpu.sync_copy(pos_hbm, pos_sh.at[pl.ds(0, T)])
        pltpu.sync_copy(pos_hbm, pos_sh.at[pl.ds(T, T)])

    plsc.subcore_barrier()

    def init_and_gather(c, b):
        t0 = lax.rem(c * CH, T)
        pltpu.sync_copy(pos_sh.at[pl.ds(t0, CH)], bufs[b])
        pltpu.async_copy(skill_emb_hbm.at[idx_s.at[c]], bufs[b], sem_s[b],
                         add=True)
        pltpu.async_copy(hard_emb_hbm.at[idx_h.at[c]], bufs[b], sem_h[b],
                         add=True)

    for k in range(LA):
        init_and_gather(k, k)

    def body(g, carry):
        for b in range(NBUF):
            c = g * NBUF + b
            bn = (b + LA) % NBUF

            @pl.when(c + LA - NBUF >= 0)
            def _():
                pltpu.make_async_copy(
                    bufs[bn], out_hbm.at[pl.ds(row0, CH), pl.ds(0, D)], sem_o[bn]).wait()

            @pl.when(c + LA < NCH)
            def _():
                init_and_gather(c + LA, bn)

            pltpu.make_async_copy(
                skill_emb_hbm.at[idx_s.at[c]], bufs[b], sem_s[b]).wait()
            pltpu.make_async_copy(
                hard_emb_hbm.at[idx_h.at[c]], bufs[b], sem_h[b]).wait()
            pltpu.async_copy(
                bufs[b], out_hbm.at[pl.ds(row0 + c * CH, CH), pl.ds(0, D)], sem_o[b])
        return carry

    lax.fori_loop(0, NCH // NBUF, body, 0)

    bl = (NCH - 1) % NBUF
    pltpu.make_async_copy(bufs[bl], out_hbm.at[pl.ds(row0, CH), pl.ds(0, D)],
                          sem_o[bl]).wait()


def kernel(skills, hardness, position_embed, skill_embed, hardness_embed):
    skills_r = skills.reshape(N // CH, CH).astype(jnp.int32)
    hardness_r = hardness.reshape(N // CH, CH).astype(jnp.int32)
    out = _embed_sc(skills_r, hardness_r, position_embed, skill_embed,
                    hardness_embed)
    return out[:, :D].reshape(B, T, D)
